# Initial kernel scaffold; baseline (speedup 1.0000x reference)
import jax
import jax.numpy as jnp
from jax.experimental import pallas as pl


def kernel(ctx, s, params):
    raise NotImplementedError("write your pallas kernel here")


# speedup vs baseline: 2.2555x; 2.2555x over previous
import functools

import jax
import jax.numpy as jnp
import numpy as np
from jax.experimental import pallas as pl
from jax.experimental.pallas import tpu as pltpu

_D = 768
_H = 12
_NB = 2
_E = 8
_K = 2
_CTX = 24
_B = 2
_T = 2048
_GH = 256
_TRUNK_K = 5
_FF = 4
_KS = [3, 5, 7, 9, 3, 5, 7, 9]
_DIL = [1, 1, 1, 1, 2, 2, 2, 2]
_DH = _D // _H
_N = _B * _T
_INTERPRET = False

_INV_SQRT2 = 0.7071067811865476

_BIG_VMEM = pltpu.CompilerParams(vmem_limit_bytes=100 * 1024 * 1024)
_PH = jax.lax.Precision.HIGHEST
_SQRT2 = float(np.sqrt(2.0))


def _bf(x):
    return x.astype(jnp.bfloat16).astype(jnp.float32)


def _gelu(x):
    return 0.5 * x * (jax.lax.erf(x / _SQRT2) + 1.0)


def _ln_in(x, g, b):
    m = jnp.mean(x, axis=-1, keepdims=True)
    v = jnp.mean((x - m) ** 2, axis=-1, keepdims=True)
    return (x - m) / jnp.sqrt(v + 1e-5) * g + b


def _shift_rows(x, off):
    if off == 0:
        return x
    rows, cols = x.shape
    z = jnp.zeros((abs(off), cols), x.dtype)
    if off > 0:
        return jnp.concatenate([x[off:], z], axis=0)
    return jnp.concatenate([z, x[:rows + off]], axis=0)


def _linear_impl(x_ref, w_ref, b_ref, *refs, act, has_ln, has_res):
    rest = list(refs)
    o_ref = rest[-1]
    acc = jnp.dot(x_ref[...], w_ref[...], preferred_element_type=jnp.float32)
    acc = acc + b_ref[...]
    if act == "gelu":
        acc = _gelu(acc)
    if has_res:
        acc = acc + rest[0][...]
    if has_ln:
        g = rest[1 if has_res else 0][...]
        be = rest[2 if has_res else 1][...]
        acc = _ln_in(acc, g, be)
    o_ref[...] = acc


def _linear(x, w, b, act=None, res=None, ln=None, bn=512, bm=None):
    n, kd = x.shape
    m = w.shape[1]
    if bm is None:
        bm = m if m <= 1024 else 768
    if ln is not None:
        bm = m
    in_specs = [
        pl.BlockSpec((bn, kd), lambda i, j: (i, 0)),
        pl.BlockSpec((kd, bm), lambda i, j: (0, j)),
        pl.BlockSpec((1, bm), lambda i, j: (0, j)),
    ]
    args = [x, w, b.reshape(1, m)]
    if res is not None:
        in_specs.append(pl.BlockSpec((bn, bm), lambda i, j: (i, j)))
        args.append(res)
    if ln is not None:
        g, be = ln
        in_specs.append(pl.BlockSpec((1, m), lambda i, j: (0, 0)))
        in_specs.append(pl.BlockSpec((1, m), lambda i, j: (0, 0)))
        args.append(g.reshape(1, m))
        args.append(be.reshape(1, m))
    return pl.pallas_call(
        functools.partial(_linear_impl, act=act, has_ln=ln is not None,
                          has_res=res is not None),
        grid=(n // bn, m // bm),
        in_specs=in_specs,
        out_specs=pl.BlockSpec((bn, bm), lambda i, j: (i, j)),
        out_shape=jax.ShapeDtypeStruct((n, m), jnp.float32),
        interpret=_INTERPRET,
    )(*args)


def _embed_impl(s_ref, w1_ref, b1_ref, w2_ref, b2_ref, ctx_ref, wc_ref,
                bc_ref, o_ref):
    sv = s_ref[...]
    a = _gelu(_bf(sv) * _bf(w1_ref[...]) + b1_ref[...])
    h = jnp.dot(a, w2_ref[...], preferred_element_type=jnp.float32)
    h = h + b2_ref[...]
    c = jnp.dot(ctx_ref[0], wc_ref[...], preferred_element_type=jnp.float32)
    o_ref[...] = h + c + bc_ref[...]


def _embed(s, se, ctx, cp, bn=512):
    s2 = s.reshape(_N, 1)
    w1 = se["W1"][:, 0].reshape(1, _D // 2)
    w2 = se["W2"].T
    wc = cp["W"].T
    ctx3 = ctx.reshape(_B, 1, _CTX)
    return pl.pallas_call(
        _embed_impl,
        grid=(_N // bn,),
        in_specs=[
            pl.BlockSpec((bn, 1), lambda i: (i, 0)),
            pl.BlockSpec((1, _D // 2), lambda i: (0, 0)),
            pl.BlockSpec((1, _D // 2), lambda i: (0, 0)),
            pl.BlockSpec((_D // 2, _D), lambda i: (0, 0)),
            pl.BlockSpec((1, _D), lambda i: (0, 0)),
            pl.BlockSpec((1, 1, _CTX), lambda i: (i * bn // _T, 0, 0)),
            pl.BlockSpec((_CTX, _D), lambda i: (0, 0)),
            pl.BlockSpec((1, _D), lambda i: (0, 0)),
        ],
        out_specs=pl.BlockSpec((bn, _D), lambda i: (i, 0)),
        out_shape=jax.ShapeDtypeStruct((_N, _D), jnp.float32),
        interpret=_INTERPRET,
    )(s2, w1, se["b1"].reshape(1, -1), w2, se["b2"].reshape(1, -1), ctx3, wc,
      cp["b"].reshape(1, -1))


def _gconv_impl(x_ref, m_ref, cb_ref, g_ref, be_ref, o_ref):
    x = x_ref[0]
    acc = jnp.broadcast_to(cb_ref[...], (_T, _D))
    for tap in range(_TRUNK_K):
        xs = _shift_rows(x, tap - _TRUNK_K // 2)
        acc = acc + jnp.dot(xs, m_ref[tap],
                            preferred_element_type=jnp.float32)
    o_ref[0] = _ln_in(x + acc, g_ref[...], be_ref[...])


def _gconv(h, bp):
    cw = bp["conv_w"]
    bd = cw.reshape(_D // 8, 8, 8, _TRUNK_K)
    eye = jnp.eye(_D // 8, dtype=jnp.float32)
    m = jnp.einsum("gojt,gh->tgjho", bd, eye).reshape(_TRUNK_K, _D, _D)
    x3 = h.reshape(_B, _T, _D)
    out = pl.pallas_call(
        _gconv_impl,
        compiler_params=_BIG_VMEM,
        grid=(_B,),
        in_specs=[
            pl.BlockSpec((1, _T, _D), lambda i: (i, 0, 0)),
            pl.BlockSpec((_TRUNK_K, _D, _D), lambda i: (0, 0, 0)),
            pl.BlockSpec((1, _D), lambda i: (0, 0)),
            pl.BlockSpec((1, _D), lambda i: (0, 0)),
            pl.BlockSpec((1, _D), lambda i: (0, 0)),
        ],
        out_specs=pl.BlockSpec((1, _T, _D), lambda i: (i, 0, 0)),
        out_shape=jax.ShapeDtypeStruct((_B, _T, _D), jnp.float32),
        interpret=_INTERPRET,
    )(x3, m, bp["conv_b"].reshape(1, -1), bp["ln1_g"].reshape(1, -1),
      bp["ln1_b"].reshape(1, -1))
    return out.reshape(_N, _D)


def _attn_impl(q_ref, k_ref, v_ref, o_ref):
    q = q_ref[0]
    k = k_ref[0]
    v = v_ref[0]
    s = jax.lax.dot_general(q, k, (((1,), (1,)), ((), ())),
                            preferred_element_type=jnp.float32)
    s = s * (1.0 / np.sqrt(_DH))
    s = s - jnp.max(s, axis=-1, keepdims=True)
    p = jnp.exp(s)
    p = p / jnp.sum(p, axis=-1, keepdims=True)
    o_ref[0] = jnp.dot(p, v, preferred_element_type=jnp.float32)


def _attention(qkv, bq=512):
    nq = _T // bq
    qkv4 = qkv.reshape(_B, _T, 3, _H, _DH).transpose(2, 0, 3, 1, 4)
    qkv4 = qkv4.reshape(3, _B * _H, _T, _DH)
    out = pl.pallas_call(
        _attn_impl,
        grid=(_B * _H, nq),
        in_specs=[
            pl.BlockSpec((1, bq, _DH), lambda hh, i: (hh, i, 0)),
            pl.BlockSpec((1, _T, _DH), lambda hh, i: (hh, 0, 0)),
            pl.BlockSpec((1, _T, _DH), lambda hh, i: (hh, 0, 0)),
        ],
        out_specs=pl.BlockSpec((1, bq, _DH), lambda hh, i: (hh, i, 0)),
        out_shape=jax.ShapeDtypeStruct((_B * _H, _T, _DH), jnp.float32),
        interpret=_INTERPRET,
    )(qkv4[0], qkv4[1], qkv4[2])
    return out.reshape(_B, _H, _T, _DH).transpose(0, 2, 1, 3).reshape(_N, _D)


def _gate_impl(x_ref, w1g_ref, b1g_ref, w2g_ref, b2g_ref, w1c_ref, b1c_ref,
               w2c_ref, b2c_ref, ow_ref, oi_ref, oc_ref):
    x = x_ref[0]
    pooled = jnp.mean(x, axis=0, keepdims=True)
    a = _gelu(jnp.dot(pooled, w1g_ref[...],
                      preferred_element_type=jnp.float32) + b1g_ref[...])
    logits = jnp.dot(a, w2g_ref[...],
                     preferred_element_type=jnp.float32) + b2g_ref[...]
    c = _gelu(jnp.dot(pooled, w1c_ref[...],
                      preferred_element_type=jnp.float32) + b1c_ref[...])
    sc = jnp.dot(c, w2c_ref[...],
                 preferred_element_type=jnp.float32) + b2c_ref[...]
    iota = jax.lax.broadcasted_iota(jnp.int32, (1, _E), 1)
    v1 = jnp.max(logits, axis=1, keepdims=True)
    i1 = jnp.min(jnp.where(logits >= v1, iota, _E + 1), axis=1, keepdims=True)
    a2 = jnp.where(iota == i1, -1e30, logits)
    v2 = jnp.max(a2, axis=1, keepdims=True)
    i2 = jnp.min(jnp.where(a2 >= v2, iota, _E + 1), axis=1, keepdims=True)
    w1 = 1.0 / (1.0 + jnp.exp(v2 - v1))
    wvec = jnp.where(iota == 0, w1, jnp.where(iota == 1, 1.0 - w1, 0.0))
    ivec = jnp.where(iota == 0, i1, jnp.where(iota == 1, i2, 0))
    ow_ref[0] = wvec
    oi_ref[0] = ivec
    oc_ref[0] = sc


def _gate(h, gp, cp):
    x3 = h.reshape(_B, _T, _D)
    w2c = jnp.zeros((_D // 2, _E), jnp.float32).at[:, :2].set(cp["W2"].T)
    b2c = jnp.zeros((1, _E), jnp.float32).at[0, :2].set(cp["b2"])
    b2g = gp["b2"].reshape(1, _E)
    full = lambda shape: pl.BlockSpec(shape, lambda i: tuple(0 for _ in shape))
    return pl.pallas_call(
        _gate_impl,
        compiler_params=_BIG_VMEM,
        grid=(_B,),
        in_specs=[
            pl.BlockSpec((1, _T, _D), lambda i: (i, 0, 0)),
            full((_D, _GH)), full((1, _GH)), full((_GH, _E)), full((1, _E)),
            full((_D, _D // 2)), full((1, _D // 2)), full((_D // 2, _E)),
            full((1, _E)),
        ],
        out_specs=[
            pl.BlockSpec((1, 1, _E), lambda i: (i, 0, 0)),
            pl.BlockSpec((1, 1, _E), lambda i: (i, 0, 0)),
            pl.BlockSpec((1, 1, _E), lambda i: (i, 0, 0)),
        ],
        out_shape=[
            jax.ShapeDtypeStruct((_B, 1, _E), jnp.float32),
            jax.ShapeDtypeStruct((_B, 1, _E), jnp.int32),
            jax.ShapeDtypeStruct((_B, 1, _E), jnp.float32),
        ],
        interpret=_INTERPRET,
    )(x3, gp["W1"].T, gp["b1"].reshape(1, -1), gp["W2"].T, b2g,
      cp["W1"].T, cp["b1"].reshape(1, -1), w2c, b2c)


_DW_TAPS = 13


def _moe_impl(idx_ref, x_ref, dwd_ref, dwb_ref, pw_ref, pwb_ref, gng_ref,
              gnb_ref, gsel_ref, w_ref, o_ref):
    b = pl.program_id(0)
    s = pl.program_id(1)
    x = x_ref[0]
    h = jnp.broadcast_to(dwb_ref[0], (_T, _D))
    for tap in range(_DW_TAPS):
        off = tap - _DW_TAPS // 2
        xs = _shift_rows(x, off)
        h = h + _bf(xs) * _bf(dwd_ref[0, tap])
    h2 = jnp.dot(h, pw_ref[0], preferred_element_type=jnp.float32)
    h2 = h2 + pwb_ref[0, 0]
    g = _gelu(h2)
    gsel = gsel_ref[...]
    s1 = jnp.sum(jnp.dot(g, gsel, preferred_element_type=jnp.float32,
                         precision=_PH), axis=0, keepdims=True)
    s2 = jnp.sum(jnp.dot(g * g, gsel, preferred_element_type=jnp.float32,
                         precision=_PH), axis=0, keepdims=True)
    cnt = float(_T * (_D // 8))
    mu = s1 / cnt
    var = s2 / cnt - mu * mu
    muc = jax.lax.dot_general(mu, gsel, (((1,), (1,)), ((), ())),
                              preferred_element_type=jnp.float32,
                              precision=_PH)
    varc = jax.lax.dot_general(var, gsel, (((1,), (1,)), ((), ())),
                               preferred_element_type=jnp.float32,
                               precision=_PH)
    y = (g - muc) / jnp.sqrt(varc + 1e-5) * gng_ref[0, 0] + gnb_ref[0, 0]
    w = w_ref[b, s]
    contrib = w * y

    @pl.when(s == 0)
    def _():
        o_ref[0] = x + contrib

    @pl.when(s > 0)
    def _():
        o_ref[0] = o_ref[0] + contrib


def _moe(h, experts, weights, idxs):
    x3 = h.reshape(_B, _T, _D)
    dwd = []
    for e in range(_E):
        k, dil = _KS[e], _DIL[e]
        dw = experts[e]["dw_w"][:, 0, :]
        dense = jnp.zeros((_DW_TAPS, _D), jnp.float32)
        for i in range(k):
            off = (i - k // 2) * dil + _DW_TAPS // 2
            dense = dense.at[off].set(dw[:, i])
        dwd.append(dense)
    dwd = jnp.stack(dwd)
    dwb = jnp.stack([experts[e]["dw_b"] for e in range(_E)]).reshape(_E, 1, _D)
    pw = jnp.stack([experts[e]["pw_w"][:, :, 0].T for e in range(_E)])
    pwb = jnp.stack([experts[e]["pw_b"] for e in range(_E)]).reshape(_E, 1, _D)
    gng = jnp.stack([experts[e]["gn_g"] for e in range(_E)]).reshape(_E, 1, _D)
    gnb = jnp.stack([experts[e]["gn_b"] for e in range(_E)]).reshape(_E, 1, _D)
    gsel = (np.arange(_D)[:, None] // (_D // 8) ==
            np.arange(8)[None, :]).astype(np.float32)
    gsel = jnp.asarray(gsel)

    grid_spec = pltpu.PrefetchScalarGridSpec(
        num_scalar_prefetch=1,
        grid=(_B, _K),
        in_specs=[
            pl.BlockSpec((1, _T, _D), lambda b, s, idx: (b, 0, 0)),
            pl.BlockSpec((1, _DW_TAPS, _D),
                         lambda b, s, idx: (idx[b * _K + s], 0, 0)),
            pl.BlockSpec((1, 1, _D), lambda b, s, idx: (idx[b * _K + s], 0, 0)),
            pl.BlockSpec((1, _D, _D), lambda b, s, idx: (idx[b * _K + s], 0, 0)),
            pl.BlockSpec((1, 1, _D), lambda b, s, idx: (idx[b * _K + s], 0, 0)),
            pl.BlockSpec((1, 1, _D), lambda b, s, idx: (idx[b * _K + s], 0, 0)),
            pl.BlockSpec((1, 1, _D), lambda b, s, idx: (idx[b * _K + s], 0, 0)),
            pl.BlockSpec((_D, 8), lambda b, s, idx: (0, 0)),
            pl.BlockSpec(memory_space=pltpu.SMEM),
        ],
        out_specs=pl.BlockSpec((1, _T, _D), lambda b, s, idx: (b, 0, 0)),
    )
    out = pl.pallas_call(
        _moe_impl,
        compiler_params=_BIG_VMEM,
        grid_spec=grid_spec,
        out_shape=jax.ShapeDtypeStruct((_B, _T, _D), jnp.float32),
        interpret=_INTERPRET,
    )(idxs, x3, dwd, dwb, pw, pwb, gng, gnb, gsel, weights)
    return out


def _heads_impl(x_ref, lng_ref, lnb_ref, w1_ref, b1_ref, w2_ref, b2_ref,
                c1_ref, c1b_ref, c2_ref, c2b_ref, ota_ref, osw_ref):
    x = x_ref[0]
    xl = _ln_in(x, lng_ref[...], lnb_ref[...])
    a1 = _gelu(jnp.dot(xl, w1_ref[...],
                       preferred_element_type=jnp.float32) + b1_ref[...])
    ta = jnp.sum(_bf(a1) * _bf(w2_ref[...]), axis=1) + b2_ref[0, 0]
    ota_ref[0, 0] = ta
    y = jnp.broadcast_to(c1b_ref[...], (_T, _D // 2))
    for tap in range(3):
        xs = _shift_rows(x, tap - 1)
        y = y + jnp.dot(xs, c1_ref[tap], preferred_element_type=jnp.float32)
    gswv = _gelu(y)
    sw = jnp.sum(_bf(gswv) * _bf(c2_ref[...]), axis=1) + c2b_ref[0, 0]
    osw_ref[0, 0] = sw


def _heads(h, hm, hs):
    x3 = h.reshape(_B, _T, _D)
    c1 = jnp.stack([hs["c1_w"][:, :, i].T for i in range(3)])
    full = lambda shape: pl.BlockSpec(shape, lambda i: tuple(0 for _ in shape))
    return pl.pallas_call(
        _heads_impl,
        compiler_params=_BIG_VMEM,
        grid=(_B,),
        in_specs=[
            pl.BlockSpec((1, _T, _D), lambda i: (i, 0, 0)),
            full((1, _D)), full((1, _D)),
            full((_D, _D // 2)), full((1, _D // 2)), full((1, _D // 2)),
            full((1, 1)),
            full((3, _D, _D // 2)), full((1, _D // 2)), full((1, _D // 2)),
            full((1, 1)),
        ],
        out_specs=[
            pl.BlockSpec((1, 1, _T), lambda i: (i, 0, 0)),
            pl.BlockSpec((1, 1, _T), lambda i: (i, 0, 0)),
        ],
        out_shape=[
            jax.ShapeDtypeStruct((_B, 1, _T), jnp.float32),
            jax.ShapeDtypeStruct((_B, 1, _T), jnp.float32),
        ],
        interpret=_INTERPRET,
    )(x3, hm["ln_g"].reshape(1, -1), hm["ln_b"].reshape(1, -1),
      hm["W1"].T, hm["b1"].reshape(1, -1), hm["W2"].reshape(1, -1),
      hm["b2"].reshape(1, 1),
      c1, hs["c1_b"].reshape(1, -1), hs["c2_w"][:, :, 0].reshape(1, -1),
      hs["c2_b"].reshape(1, 1))


def kernel(ctx, s, params):
    p = params
    h = _embed(s, p["s_embed"], ctx, p["ctx_proj"])
    for bp in p["trunk"]:
        x1 = _gconv(h, bp)
        qkv = _linear(x1, bp["attn_in_w"].T, bp["attn_in_b"])
        attn = _attention(qkv)
        x2 = _linear(attn, bp["attn_out_w"].T, bp["attn_out_b"], res=x1,
                     ln=(bp["ln2_g"], bp["ln2_b"]))
        f1 = _linear(x2, bp["ff_W1"].T, bp["ff_b1"], act="gelu")
        h = _linear(f1, bp["ff_W2"].T, bp["ff_b2"], res=x2,
                    ln=(bp["ln3_g"], bp["ln3_b"]))
    w3, i3, sc3 = _gate(h, p["gate"], p["head_sc"])
    weights = w3[:, 0, :_K]
    idxs = i3[:, 0, :_K].reshape(_B * _K)
    hf = _moe(h, p["experts"], weights, idxs)
    ta, sw = _heads(hf.reshape(_N, _D), p["head_main"], p["head_sw"])
    return (ta[:, 0, :], sw[:, 0, :], sc3[:, 0, 0], sc3[:, 0, 1], weights)
